# Initial kernel scaffold; baseline (speedup 1.0000x reference)
import jax
import jax.numpy as jnp
from jax.experimental import pallas as pl


def kernel(unknown, known, unknow_feats, known_feats, W0, gamma0, beta0):
    raise NotImplementedError("write your pallas kernel here")


# speedup vs baseline: 28.6252x; 28.6252x over previous
import jax
import jax.numpy as jnp
from jax.experimental import pallas as pl

_B, _N, _M = 4, 8192, 2048
_C1, _C2 = 64, 128
_C_OUT = 128
_TILE = 256
_TILE2 = 2048
_IDX_MASK = 0x7FF
_BIG = 0x7FFFFFFF


def _pass1_body(u_ref, k_ref, kf_ref, uf_ref, w0a_ref, w0b_ref,
                y_ref, acc_ref):
    u = u_ref[0]
    kn = k_ref[0]

    cross = jnp.dot(u, kn, preferred_element_type=jnp.float32)
    u2 = jnp.sum(u * u, axis=1, keepdims=True)
    k2 = jnp.sum(kn * kn, axis=0, keepdims=True)
    d2 = jnp.maximum(u2 + k2 - 2.0 * cross, 0.0)

    bits = jax.lax.bitcast_convert_type(d2, jnp.int32)
    col = jax.lax.broadcasted_iota(jnp.int32, (_TILE, _M), 1)
    big = jnp.int32(_BIG)

    def extract(cur):
        m = jnp.min(cur, axis=1, keepdims=True)
        i = jnp.min(jnp.where(cur == m, col, big), axis=1, keepdims=True)
        dist = jax.lax.bitcast_convert_type(m, jnp.float32)
        return i, dist, jnp.where(col == i, big, cur)

    i1, d1, bits = extract(bits)
    i2, dd2, bits = extract(bits)
    i3, d3, _ = extract(bits)
    r1 = 1.0 / (d1 + 1e-8)
    r2 = 1.0 / (dd2 + 1e-8)
    r3 = 1.0 / (d3 + 1e-8)
    norm = r1 + r2 + r3
    w1, w2, w3 = r1 / norm, r2 / norm, r3 / norm

    zero = jnp.float32(0.0)
    wmat = (jnp.where(col == i1, w1, zero)
            + jnp.where(col == i2, w2, zero)
            + jnp.where(col == i3, w3, zero))

    kf = kf_ref[0]
    interp = jax.lax.dot_general(
        kf, wmat, (((1,), (1,)), ((), ())),
        preferred_element_type=jnp.float32)

    uf = uf_ref[0]
    y = (jnp.dot(w0a_ref[...], interp, preferred_element_type=jnp.float32)
         + jnp.dot(w0b_ref[...], uf, preferred_element_type=jnp.float32))

    y_ref[0] = y

    s = jnp.sum(y, axis=1, keepdims=True)
    s2 = jnp.sum(y * y, axis=1, keepdims=True)
    lane = jax.lax.broadcasted_iota(jnp.int32, (_C_OUT, 128), 1)
    contrib = jnp.where(lane == 0, s, zero) + jnp.where(lane == 1, s2, zero)

    @pl.when(jnp.logical_and(pl.program_id(0) == 0, pl.program_id(1) == 0))
    def _():
        acc_ref[...] = jnp.zeros_like(acc_ref)

    acc_ref[...] += contrib


def _pass2_body(y_ref, acc_ref, gamma_ref, beta_ref, out_ref):
    s = acc_ref[:, 0:1]
    s2 = acc_ref[:, 1:2]
    cnt = jnp.float32(_B * _N)
    mean = s / cnt
    var = s2 / cnt - mean * mean
    scale = gamma_ref[...] * jax.lax.rsqrt(var + 1e-5)
    shift = beta_ref[...] - mean * scale
    out_ref[0] = jnp.maximum(y_ref[0] * scale + shift, 0.0)


def kernel(unknown, known, unknow_feats, known_feats, W0, gamma0, beta0):
    pad_u = jnp.zeros((_B, _N, 5), jnp.float32)
    ub = jnp.concatenate([unknown, pad_u], axis=2)
    kt = jnp.transpose(known, (0, 2, 1))
    pad_k = jnp.zeros((_B, 5, _M), jnp.float32)
    kb = jnp.concatenate([kt, pad_k], axis=1)
    w0a = W0[:, :_C2]
    w0b = W0[:, _C2:]

    grid1 = (_B, _N // _TILE)
    y, acc = pl.pallas_call(
        _pass1_body,
        grid=grid1,
        in_specs=[
            pl.BlockSpec((1, _TILE, 8), lambda b, t: (b, t, 0)),
            pl.BlockSpec((1, 8, _M), lambda b, t: (b, 0, 0)),
            pl.BlockSpec((1, _C2, _M), lambda b, t: (b, 0, 0)),
            pl.BlockSpec((1, _C1, _TILE), lambda b, t: (b, 0, t)),
            pl.BlockSpec((_C_OUT, _C2), lambda b, t: (0, 0)),
            pl.BlockSpec((_C_OUT, _C1), lambda b, t: (0, 0)),
        ],
        out_specs=[
            pl.BlockSpec((1, _C_OUT, _TILE), lambda b, t: (b, 0, t)),
            pl.BlockSpec((_C_OUT, 128), lambda b, t: (0, 0)),
        ],
        out_shape=[
            jax.ShapeDtypeStruct((_B, _C_OUT, _N), jnp.float32),
            jax.ShapeDtypeStruct((_C_OUT, 128), jnp.float32),
        ],
    )(ub, kb, known_feats, unknow_feats, w0a, w0b)

    grid2 = (_B, _N // _TILE2)
    out = pl.pallas_call(
        _pass2_body,
        grid=grid2,
        in_specs=[
            pl.BlockSpec((1, _C_OUT, _TILE2), lambda b, t: (b, 0, t)),
            pl.BlockSpec((_C_OUT, 128), lambda b, t: (0, 0)),
            pl.BlockSpec((_C_OUT, 1), lambda b, t: (0, 0)),
            pl.BlockSpec((_C_OUT, 1), lambda b, t: (0, 0)),
        ],
        out_specs=pl.BlockSpec((1, _C_OUT, _TILE2), lambda b, t: (b, 0, t)),
        out_shape=jax.ShapeDtypeStruct((_B, _C_OUT, _N), jnp.float32),
    )(y, acc, gamma0.reshape(_C_OUT, 1), beta0.reshape(_C_OUT, 1))

    return out


# speedup vs baseline: 34.5093x; 1.2056x over previous
import jax
import jax.numpy as jnp
from jax.experimental import pallas as pl

_B, _N, _M = 4, 8192, 2048
_C1, _C2 = 64, 128
_C_OUT = 128
_TILE = 256
_TILE2 = 2048
_BIGF = 3.0e38


def _pass1_body(u_ref, k_ref, kf_ref, uf_ref, w0a_ref, w0b_ref,
                y_ref, acc_ref):
    u = u_ref[0]
    kn = k_ref[0]

    k2row = jnp.sum(kn * kn, axis=0, keepdims=True)
    rowi = jax.lax.broadcasted_iota(jnp.int32, (8, _M), 0)
    kaug = jnp.where(rowi == 3, jnp.broadcast_to(k2row, (8, _M)), kn)
    e2 = jnp.dot(u, kaug, preferred_element_type=jnp.float32)

    colf = jax.lax.broadcasted_iota(
        jnp.int32, (_TILE, _M), 1).astype(jnp.float32)
    bigf = jnp.float32(_BIGF)

    def extract(cur):
        v = jnp.min(cur, axis=1, keepdims=True)
        i = jnp.min(jnp.where(cur == v, colf, bigf), axis=1, keepdims=True)
        return v, colf == i

    v1, eq1 = extract(e2)
    cur = jnp.where(eq1, bigf, e2)
    v2, eq2 = extract(cur)
    cur = jnp.where(eq2, bigf, cur)
    v3, eq3 = extract(cur)

    u2 = 0.25 * (jnp.sum(u * u, axis=1, keepdims=True) - 1.0)
    r1 = 1.0 / (jnp.maximum(v1 + u2, 0.0) + 1e-8)
    r2 = 1.0 / (jnp.maximum(v2 + u2, 0.0) + 1e-8)
    r3 = 1.0 / (jnp.maximum(v3 + u2, 0.0) + 1e-8)
    norm = r1 + r2 + r3
    w1, w2, w3 = r1 / norm, r2 / norm, r3 / norm

    zero = jnp.float32(0.0)
    wmat = jnp.where(eq1, w1, jnp.where(eq2, w2, jnp.where(eq3, w3, zero)))

    kf = kf_ref[0]
    interp = jax.lax.dot_general(
        kf, wmat, (((1,), (1,)), ((), ())),
        preferred_element_type=jnp.float32)

    uf = uf_ref[0]
    y = (jnp.dot(w0a_ref[...], interp, preferred_element_type=jnp.float32)
         + jnp.dot(w0b_ref[...], uf, preferred_element_type=jnp.float32))

    y_ref[0] = y

    s = jnp.sum(y, axis=1, keepdims=True)
    s2 = jnp.sum(y * y, axis=1, keepdims=True)
    lane = jax.lax.broadcasted_iota(jnp.int32, (_C_OUT, 128), 1)
    contrib = jnp.where(lane == 0, s, zero) + jnp.where(lane == 1, s2, zero)

    @pl.when(jnp.logical_and(pl.program_id(0) == 0, pl.program_id(1) == 0))
    def _():
        acc_ref[...] = jnp.zeros_like(acc_ref)

    acc_ref[...] += contrib


def _pass2_body(y_ref, acc_ref, gamma_ref, beta_ref, out_ref):
    s = acc_ref[:, 0:1]
    s2 = acc_ref[:, 1:2]
    cnt = jnp.float32(_B * _N)
    mean = s / cnt
    var = s2 / cnt - mean * mean
    scale = gamma_ref[...] * jax.lax.rsqrt(var + 1e-5)
    shift = beta_ref[...] - mean * scale
    out_ref[0] = jnp.maximum(y_ref[0] * scale + shift, 0.0)


def kernel(unknown, known, unknow_feats, known_feats, W0, gamma0, beta0):
    ones_u = jnp.ones((_B, _N, 1), jnp.float32)
    pad_u = jnp.zeros((_B, _N, 4), jnp.float32)
    ub = jnp.concatenate([-2.0 * unknown, ones_u, pad_u], axis=2)
    kt = jnp.transpose(known, (0, 2, 1))
    pad_k = jnp.zeros((_B, 5, _M), jnp.float32)
    kb = jnp.concatenate([kt, pad_k], axis=1)
    w0a = W0[:, :_C2]
    w0b = W0[:, _C2:]

    grid1 = (_B, _N // _TILE)
    y, acc = pl.pallas_call(
        _pass1_body,
        grid=grid1,
        in_specs=[
            pl.BlockSpec((1, _TILE, 8), lambda b, t: (b, t, 0)),
            pl.BlockSpec((1, 8, _M), lambda b, t: (b, 0, 0)),
            pl.BlockSpec((1, _C2, _M), lambda b, t: (b, 0, 0)),
            pl.BlockSpec((1, _C1, _TILE), lambda b, t: (b, 0, t)),
            pl.BlockSpec((_C_OUT, _C2), lambda b, t: (0, 0)),
            pl.BlockSpec((_C_OUT, _C1), lambda b, t: (0, 0)),
        ],
        out_specs=[
            pl.BlockSpec((1, _C_OUT, _TILE), lambda b, t: (b, 0, t)),
            pl.BlockSpec((_C_OUT, 128), lambda b, t: (0, 0)),
        ],
        out_shape=[
            jax.ShapeDtypeStruct((_B, _C_OUT, _N), jnp.float32),
            jax.ShapeDtypeStruct((_C_OUT, 128), jnp.float32),
        ],
    )(ub, kb, known_feats, unknow_feats, w0a, w0b)

    grid2 = (_B, _N // _TILE2)
    out = pl.pallas_call(
        _pass2_body,
        grid=grid2,
        in_specs=[
            pl.BlockSpec((1, _C_OUT, _TILE2), lambda b, t: (b, 0, t)),
            pl.BlockSpec((_C_OUT, 128), lambda b, t: (0, 0)),
            pl.BlockSpec((_C_OUT, 1), lambda b, t: (0, 0)),
            pl.BlockSpec((_C_OUT, 1), lambda b, t: (0, 0)),
        ],
        out_specs=pl.BlockSpec((1, _C_OUT, _TILE2), lambda b, t: (b, 0, t)),
        out_shape=jax.ShapeDtypeStruct((_B, _C_OUT, _N), jnp.float32),
    )(y, acc, gamma0.reshape(_C_OUT, 1), beta0.reshape(_C_OUT, 1))

    return out
